# Initial kernel scaffold; baseline (speedup 1.0000x reference)
import jax
import jax.numpy as jnp
from jax.experimental import pallas as pl


def kernel(batch, logits, data):
    raise NotImplementedError("write your pallas kernel here")


# speedup vs baseline: 6.2306x; 6.2306x over previous
import functools

import jax
import jax.numpy as jnp
from jax.experimental import pallas as pl
from jax.experimental.pallas import tpu as pltpu

N = 100000
D = 128
S = 16
BLK = 2000


def _loss_kernel(batch_ref, logits_ref, data_ref, out_ref, acc_sum, acc_cnt):
    i = pl.program_id(0)

    @pl.when(i == 0)
    def _init():
        acc_sum[...] = jnp.zeros_like(acc_sum)
        acc_cnt[...] = jnp.zeros_like(acc_cnt)

    diff = logits_ref[...] - data_ref[...]
    rowsum = jnp.sum(diff * diff, axis=1)
    bids = batch_ref[0, 0, :]
    seg = jax.lax.broadcasted_iota(jnp.int32, (S, BLK), 0)
    mask = bids[None, :] == seg
    part_sum = jnp.sum(jnp.where(mask, rowsum[None, :], 0.0), axis=1, keepdims=True)
    part_cnt = jnp.sum(mask.astype(jnp.float32), axis=1, keepdims=True)
    acc_sum[...] += part_sum
    acc_cnt[...] += part_cnt

    @pl.when(i == pl.num_programs(0) - 1)
    def _finish():
        means = acc_sum[...] / jnp.maximum(acc_cnt[...], 1.0)
        out_ref[...] = (jnp.sum(means) / (S * D)).reshape(1, 1)


@jax.jit
def _loss_scalar(batch, logits, data):
    nblk = N // BLK
    batch3 = batch.astype(jnp.int32).reshape(nblk, 1, BLK)
    out = pl.pallas_call(
        _loss_kernel,
        grid=(nblk,),
        in_specs=[
            pl.BlockSpec((1, 1, BLK), lambda i: (i, 0, 0)),
            pl.BlockSpec((BLK, D), lambda i: (i, 0)),
            pl.BlockSpec((BLK, D), lambda i: (i, 0)),
        ],
        out_specs=pl.BlockSpec((1, 1), lambda i: (0, 0)),
        out_shape=jax.ShapeDtypeStruct((1, 1), jnp.float32),
        scratch_shapes=[
            pltpu.VMEM((S, 1), jnp.float32),
            pltpu.VMEM((S, 1), jnp.float32),
        ],
    )(batch3, logits, data)
    return out[0, 0]


def kernel(batch, logits, data):
    return (_loss_scalar(batch, logits, data), logits)


# speedup vs baseline: 8.1723x; 1.3116x over previous
import functools

import jax
import jax.numpy as jnp
from jax.experimental import pallas as pl
from jax.experimental.pallas import tpu as pltpu

N = 100000
D = 128
S = 16
BLK = 10000


def _loss_kernel(batch_ref, logits_ref, data_ref, out_ref, acc_sum, acc_cnt):
    i = pl.program_id(0)

    @pl.when(i == 0)
    def _init():
        acc_sum[...] = jnp.zeros_like(acc_sum)
        acc_cnt[...] = jnp.zeros_like(acc_cnt)

    diff = logits_ref[...] - data_ref[...]
    rowsum = jnp.sum(diff * diff, axis=1)
    bids = batch_ref[0, 0, :]
    seg = jax.lax.broadcasted_iota(jnp.int32, (S, BLK), 0)
    mask = bids[None, :] == seg
    part_sum = jnp.sum(jnp.where(mask, rowsum[None, :], 0.0), axis=1, keepdims=True)
    part_cnt = jnp.sum(mask.astype(jnp.float32), axis=1, keepdims=True)
    acc_sum[...] += part_sum
    acc_cnt[...] += part_cnt

    @pl.when(i == pl.num_programs(0) - 1)
    def _finish():
        means = acc_sum[...] / jnp.maximum(acc_cnt[...], 1.0)
        out_ref[...] = (jnp.sum(means) / (S * D)).reshape(1, 1)


@jax.jit
def _loss_scalar(batch, logits, data):
    nblk = N // BLK
    batch3 = batch.astype(jnp.int32).reshape(nblk, 1, BLK)
    out = pl.pallas_call(
        _loss_kernel,
        grid=(nblk,),
        in_specs=[
            pl.BlockSpec((1, 1, BLK), lambda i: (i, 0, 0)),
            pl.BlockSpec((BLK, D), lambda i: (i, 0)),
            pl.BlockSpec((BLK, D), lambda i: (i, 0)),
        ],
        out_specs=pl.BlockSpec((1, 1), lambda i: (0, 0)),
        out_shape=jax.ShapeDtypeStruct((1, 1), jnp.float32),
        scratch_shapes=[
            pltpu.VMEM((S, 1), jnp.float32),
            pltpu.VMEM((S, 1), jnp.float32),
        ],
    )(batch3, logits, data)
    return out[0, 0]


def kernel(batch, logits, data):
    return (_loss_scalar(batch, logits, data), logits)
